# Initial kernel scaffold; baseline (speedup 1.0000x reference)
import jax
import jax.numpy as jnp
from jax.experimental import pallas as pl


def kernel(audio_semantic, video_semantic, embedding, ema_count, ema_weight, epoch):
    raise NotImplementedError("write your pallas kernel here")


# speedup vs baseline: 1.3722x; 1.3722x over previous
import functools
import math

import jax
import jax.numpy as jnp
from jax.experimental import pallas as pl
from jax.experimental.pallas import tpu as pltpu

EPS = 1e-05
DECAY = 0.99

_HI = jax.lax.Precision.HIGHEST


def _dot(a, b, dims, precision=None):
    return jax.lax.dot_general(a, b, (dims, ((), ())), precision=precision)


def _fused_kernel(
    a_ref, v_ref, emb_ref, embt_ref, ec_ref, ewt_ref,
    q_ref, ent_ref, lcm_ref, ac_ref, vc_ref, eq_ref,
    counts, dwt, ph, cons, modes,
    *, b_total, t, d, m,
):
    s = pl.program_id(0)
    b = pl.program_id(1)
    max_ent = math.log(m)

    @pl.when(jnp.logical_and(s == 0, b == 0))
    def _init():
        counts[...] = jnp.zeros_like(counts)
        dwt[...] = jnp.zeros_like(dwt)
        cons[0, 0] = 0.0
        cons[1, 0] = 0.0

    emb = emb_ref[...]
    embt = embt_ref[...]
    esq = jnp.sum(embt * embt, axis=0, keepdims=True)
    iota_m = jax.lax.broadcasted_iota(jnp.int32, (t, m), 1)

    x_a = a_ref[0]
    x_v = v_ref[0]
    x_sum = x_a + x_v
    x = jnp.where(s == 0, x_a, x_v)

    xsq = jnp.sum(x * x, axis=1, keepdims=True)
    mm = _dot(x, embt, (((1,), (0,))))
    dist = (esq + xsq) - 2.0 * mm

    dmin = jnp.min(dist, axis=1, keepdims=True)
    idx = jnp.min(jnp.where(dist == dmin, iota_m, m), axis=1,
                  keepdims=True)

    z = -jnp.sqrt(dist)
    zmax = jnp.max(z, axis=1, keepdims=True)
    ez = jnp.exp(z - zmax)
    p = ez / jnp.sum(ez, axis=1, keepdims=True)
    ent = -jnp.sum(p * jnp.log(p + 1e-05), axis=1, keepdims=True)
    adj = 1.0 - ent / max_ent

    ph_row = jnp.mean(p, axis=0, keepdims=True)
    ph[pl.ds(s * b_total + b, 1), :] = ph_row
    cons[s, 0] += jnp.sum(jnp.abs(p - ph_row))

    onehot = jnp.where(iota_m == idx, 1.0, 0.0)
    q_ref[0, 0] = _dot(onehot, emb, (((1,), (0,))), precision=_HI)
    scaled = adj * onehot
    ones_row = jnp.ones((1, t), jnp.float32)
    counts[pl.ds(s, 1), :] += _dot(ones_row, scaled, (((1,), (0,))),
                                   precision=_HI)
    dwt[pl.ds(s * d, d), :] += _dot(x_sum, scaled, (((0,), (0,))),
                                    precision=_HI)

    cnt = _dot(ones_row, onehot, (((1,), (0,))), precision=_HI)
    cmax = jnp.max(cnt)
    iota_row = jax.lax.broadcasted_iota(jnp.int32, (1, m), 1)
    modes[s, b] = jnp.min(jnp.where(cnt == cmax, iota_row, m))

    @pl.when(jnp.logical_and(s == 1, b == b_total - 1))
    def _epilogue():
        ec = ec_ref[...]
        ewt = ewt_ref[...]
        counts_a = counts[pl.ds(0, 1), :]
        counts_v = counts[pl.ds(1, 1), :]
        c1 = DECAY * ec + (1.0 - DECAY) * counts_v
        n1 = jnp.sum(c1)
        c1 = (c1 + EPS) / (n1 + m * EPS) * n1
        w1 = DECAY * ewt + 0.5 * (1.0 - DECAY) * dwt[pl.ds(d, d), :]
        c2 = DECAY * c1 + (1.0 - DECAY) * counts_a
        n2 = jnp.sum(c2)
        c2 = (c2 + EPS) / (n2 + m * EPS) * n2
        w2 = DECAY * w1 + 0.5 * (1.0 - DECAY) * dwt[pl.ds(0, d), :]
        ent_ref[...] = w2 / c2

        pha = ph[pl.ds(0, b_total), :]
        phv = ph[pl.ds(b_total, b_total), :]
        la = jnp.log(pha + 1e-10)
        lv = jnp.log(phv + 1e-10)
        sc = (_dot(pha, lv, (((1,), (1,))), precision=_HI)
              + _dot(phv, la, (((1,), (1,))), precision=_HI))
        scmax = jnp.max(-sc)
        es = jnp.exp(sc + scmax)
        row = jax.lax.broadcasted_iota(jnp.int32, (b_total, b_total), 0)
        col = jax.lax.broadcasted_iota(jnp.int32, (b_total, b_total), 1)
        diag = jnp.sum(jnp.where(row == col, es, 0.0), axis=1, keepdims=True)
        es_sum = jnp.sum(es, axis=1, keepdims=True)
        lcmcm = -jnp.mean(jnp.log(diag / (es_sum + EPS)))

        eq = 0
        for i in range(b_total):
            eq += jnp.where(modes[0, i] == modes[1, i], 1, 0)

        lcm_ref[...] = jnp.full((1, 1), lcmcm, jnp.float32)
        ac_ref[...] = jnp.full((1, 1), cons[0, 0] / b_total, jnp.float32)
        vc_ref[...] = jnp.full((1, 1), cons[1, 0] / b_total, jnp.float32)
        eq_ref[...] = jnp.full((1, 1), eq, jnp.int32)


def kernel(audio_semantic, video_semantic, embedding, ema_count, ema_weight,
           epoch):
    b, t, d = audio_semantic.shape
    m = embedding.shape[0]

    fused = functools.partial(_fused_kernel, b_total=b, t=t, d=d, m=m)
    blk = lambda s, i: (i, 0, 0)
    full2 = lambda s, i: (0, 0)

    out_shape = [
        jax.ShapeDtypeStruct((2, b, t, d), jnp.float32),
        jax.ShapeDtypeStruct((d, m), jnp.float32),
        jax.ShapeDtypeStruct((1, 1), jnp.float32),
        jax.ShapeDtypeStruct((1, 1), jnp.float32),
        jax.ShapeDtypeStruct((1, 1), jnp.float32),
        jax.ShapeDtypeStruct((1, 1), jnp.int32),
    ]
    in_specs = [
        pl.BlockSpec((1, t, d), blk),
        pl.BlockSpec((1, t, d), blk),
        pl.BlockSpec((m, d), full2),
        pl.BlockSpec((d, m), full2),
        pl.BlockSpec((1, m), full2),
        pl.BlockSpec((d, m), full2),
    ]
    out_specs = [
        pl.BlockSpec((1, 1, t, d), lambda s, i: (s, i, 0, 0)),
        pl.BlockSpec((d, m), full2),
        pl.BlockSpec((1, 1), full2),
        pl.BlockSpec((1, 1), full2),
        pl.BlockSpec((1, 1), full2),
        pl.BlockSpec((1, 1), full2),
    ]
    scratch_shapes = [
        pltpu.VMEM((2, m), jnp.float32),
        pltpu.VMEM((2 * d, m), jnp.float32),
        pltpu.VMEM((2 * b, m), jnp.float32),
        pltpu.SMEM((2, 1), jnp.float32),
        pltpu.SMEM((2, b), jnp.int32),
    ]

    outs = pl.pallas_call(
        fused,
        grid=(2, b),
        in_specs=in_specs,
        out_specs=out_specs,
        out_shape=out_shape,
        scratch_shapes=scratch_shapes,
        compiler_params=pltpu.CompilerParams(
            dimension_semantics=("arbitrary", "arbitrary"),
        ),
    )(
        audio_semantic, video_semantic, embedding, embedding.T,
        ema_count.reshape(1, m), ema_weight.T,
    )
    q, ent_t, lcm, a_cons, v_cons, eq = outs
    return (q[0], q[1], ent_t.T, lcm[0, 0], a_cons[0, 0], v_cons[0, 0],
            eq[0, 0])


# speedup vs baseline: 2.6017x; 1.8960x over previous
import functools
import math

import jax
import jax.numpy as jnp
from jax.experimental import pallas as pl
from jax.experimental.pallas import tpu as pltpu

EPS = 1e-05
DECAY = 0.99

_HI = jax.lax.Precision.HIGHEST


def _dot(a, b, dims, precision=None):
    return jax.lax.dot_general(a, b, (dims, ((), ())), precision=precision,
                               preferred_element_type=jnp.float32)


def _fused_kernel(
    a_ref, v_ref, emb_ref, embt_ref, ec_ref, ewt_ref,
    q_ref, ent_ref, lcm_ref, ac_ref, vc_ref, eq_ref,
    counts, dwt, ph, cons, modes,
    *, b_total, t, d, m,
):
    s = pl.program_id(0)
    b = pl.program_id(1)
    max_ent = math.log(m)

    @pl.when(jnp.logical_and(s == 0, b == 0))
    def _init():
        counts[...] = jnp.zeros_like(counts)
        dwt[...] = jnp.zeros_like(dwt)
        cons[0, 0] = 0.0
        cons[1, 0] = 0.0

    emb = emb_ref[...]
    embt = embt_ref[...]
    esq = jnp.sum(embt * embt, axis=0, keepdims=True)
    iota_m = jax.lax.broadcasted_iota(jnp.int32, (t, m), 1)

    x_a = a_ref[0]
    x_v = v_ref[0]
    x_sum = x_a + x_v
    x = jnp.where(s == 0, x_a, x_v)

    xsq = jnp.sum(x * x, axis=1, keepdims=True)
    mm = _dot(x, embt, (((1,), (0,))))
    dist = (esq + xsq) - 2.0 * mm

    dmin = jnp.min(dist, axis=1, keepdims=True)
    idx = jnp.min(jnp.where(dist == dmin, iota_m, m), axis=1,
                  keepdims=True)

    z = -jnp.sqrt(dist)
    zmax = jnp.max(z, axis=1, keepdims=True)
    ez = jnp.exp(z - zmax)
    p = ez / jnp.sum(ez, axis=1, keepdims=True)
    ent = -jnp.sum(p * jnp.log(p + 1e-05), axis=1, keepdims=True)
    adj = 1.0 - ent / max_ent

    ph_row = jnp.mean(p, axis=0, keepdims=True)
    ph[pl.ds(s * b_total + b, 1), :] = ph_row
    cons[s, 0] += jnp.sum(jnp.abs(p - ph_row))

    bf16 = jnp.bfloat16
    onehot = jnp.where(iota_m == idx, 1.0, 0.0).astype(bf16)
    q_ref[0, 0] = _dot(onehot, emb.astype(bf16), (((1,), (0,))))
    ones_row = jnp.ones((1, t), bf16)
    counts[pl.ds(s, 1), :] += _dot(adj.astype(bf16), onehot, (((0,), (0,))))
    dwt[pl.ds(s * d, d), :] += _dot((adj * x_sum).astype(bf16), onehot,
                                    (((0,), (0,))))

    cnt = _dot(ones_row, onehot, (((1,), (0,))))
    cmax = jnp.max(cnt)
    iota_row = jax.lax.broadcasted_iota(jnp.int32, (1, m), 1)
    modes[s, b] = jnp.min(jnp.where(cnt == cmax, iota_row, m))

    @pl.when(jnp.logical_and(s == 1, b == b_total - 1))
    def _epilogue():
        ec = ec_ref[...]
        ewt = ewt_ref[...]
        counts_a = counts[pl.ds(0, 1), :]
        counts_v = counts[pl.ds(1, 1), :]
        c1 = DECAY * ec + (1.0 - DECAY) * counts_v
        n1 = jnp.sum(c1)
        c1 = (c1 + EPS) / (n1 + m * EPS) * n1
        w1 = DECAY * ewt + 0.5 * (1.0 - DECAY) * dwt[pl.ds(d, d), :]
        c2 = DECAY * c1 + (1.0 - DECAY) * counts_a
        n2 = jnp.sum(c2)
        c2 = (c2 + EPS) / (n2 + m * EPS) * n2
        w2 = DECAY * w1 + 0.5 * (1.0 - DECAY) * dwt[pl.ds(0, d), :]
        ent_ref[...] = w2 / c2

        pha = ph[pl.ds(0, b_total), :]
        phv = ph[pl.ds(b_total, b_total), :]
        la = jnp.log(pha + 1e-10)
        lv = jnp.log(phv + 1e-10)
        sc = (_dot(pha, lv, (((1,), (1,))), precision=_HI)
              + _dot(phv, la, (((1,), (1,))), precision=_HI))
        scmax = jnp.max(-sc)
        es = jnp.exp(sc + scmax)
        row = jax.lax.broadcasted_iota(jnp.int32, (b_total, b_total), 0)
        col = jax.lax.broadcasted_iota(jnp.int32, (b_total, b_total), 1)
        diag = jnp.sum(jnp.where(row == col, es, 0.0), axis=1, keepdims=True)
        es_sum = jnp.sum(es, axis=1, keepdims=True)
        lcmcm = -jnp.mean(jnp.log(diag / (es_sum + EPS)))

        eq = 0
        for i in range(b_total):
            eq += jnp.where(modes[0, i] == modes[1, i], 1, 0)

        lcm_ref[...] = jnp.full((1, 1), lcmcm, jnp.float32)
        ac_ref[...] = jnp.full((1, 1), cons[0, 0] / b_total, jnp.float32)
        vc_ref[...] = jnp.full((1, 1), cons[1, 0] / b_total, jnp.float32)
        eq_ref[...] = jnp.full((1, 1), eq, jnp.int32)


def kernel(audio_semantic, video_semantic, embedding, ema_count, ema_weight,
           epoch):
    b, t, d = audio_semantic.shape
    m = embedding.shape[0]

    fused = functools.partial(_fused_kernel, b_total=b, t=t, d=d, m=m)
    blk = lambda s, i: (i, 0, 0)
    full2 = lambda s, i: (0, 0)

    out_shape = [
        jax.ShapeDtypeStruct((2, b, t, d), jnp.float32),
        jax.ShapeDtypeStruct((d, m), jnp.float32),
        jax.ShapeDtypeStruct((1, 1), jnp.float32),
        jax.ShapeDtypeStruct((1, 1), jnp.float32),
        jax.ShapeDtypeStruct((1, 1), jnp.float32),
        jax.ShapeDtypeStruct((1, 1), jnp.int32),
    ]
    in_specs = [
        pl.BlockSpec((1, t, d), blk),
        pl.BlockSpec((1, t, d), blk),
        pl.BlockSpec((m, d), full2),
        pl.BlockSpec((d, m), full2),
        pl.BlockSpec((1, m), full2),
        pl.BlockSpec((d, m), full2),
    ]
    out_specs = [
        pl.BlockSpec((1, 1, t, d), lambda s, i: (s, i, 0, 0)),
        pl.BlockSpec((d, m), full2),
        pl.BlockSpec((1, 1), full2),
        pl.BlockSpec((1, 1), full2),
        pl.BlockSpec((1, 1), full2),
        pl.BlockSpec((1, 1), full2),
    ]
    scratch_shapes = [
        pltpu.VMEM((2, m), jnp.float32),
        pltpu.VMEM((2 * d, m), jnp.float32),
        pltpu.VMEM((2 * b, m), jnp.float32),
        pltpu.SMEM((2, 1), jnp.float32),
        pltpu.SMEM((2, b), jnp.int32),
    ]

    outs = pl.pallas_call(
        fused,
        grid=(2, b),
        in_specs=in_specs,
        out_specs=out_specs,
        out_shape=out_shape,
        scratch_shapes=scratch_shapes,
        compiler_params=pltpu.CompilerParams(
            dimension_semantics=("arbitrary", "arbitrary"),
        ),
    )(
        audio_semantic, video_semantic, embedding, embedding.T,
        ema_count.reshape(1, m), ema_weight.T,
    )
    q, ent_t, lcm, a_cons, v_cons, eq = outs
    return (q[0], q[1], ent_t.T, lcm[0, 0], a_cons[0, 0], v_cons[0, 0],
            eq[0, 0])


# speedup vs baseline: 2.6701x; 1.0263x over previous
import functools
import math

import jax
import jax.numpy as jnp
from jax.experimental import pallas as pl
from jax.experimental.pallas import tpu as pltpu

EPS = 1e-05
DECAY = 0.99

_HI = jax.lax.Precision.HIGHEST


def _dot(a, b, dims, precision=None):
    return jax.lax.dot_general(a, b, (dims, ((), ())), precision=precision,
                               preferred_element_type=jnp.float32)


def _fused_kernel(
    a_ref, v_ref, emb_ref, embt_ref, ec_ref, ewt_ref,
    q_ref, ent_ref, lcm_ref, ac_ref, vc_ref, eq_ref,
    counts, dwt, ph, cons, modes, esq_s,
    *, b_total, t, d, m,
):
    s = pl.program_id(0)
    b = pl.program_id(1)
    max_ent = math.log(m)

    @pl.when(jnp.logical_and(s == 0, b == 0))
    def _init():
        counts[...] = jnp.zeros_like(counts)
        dwt[...] = jnp.zeros_like(dwt)
        cons[0, 0] = 0.0
        cons[1, 0] = 0.0
        embt0 = embt_ref[...]
        esq_s[...] = jnp.sum(embt0 * embt0, axis=0, keepdims=True)

    emb = emb_ref[...]
    embt = embt_ref[...]
    esq = esq_s[...]
    iota_m = jax.lax.broadcasted_iota(jnp.int32, (t, m), 1)

    x_a = a_ref[0]
    x_v = v_ref[0]
    x_sum = x_a + x_v
    x = jnp.where(s == 0, x_a, x_v)

    xsq = jnp.sum(x * x, axis=1, keepdims=True)
    mm = _dot(x, embt, (((1,), (0,))))
    dist = (esq + xsq) - 2.0 * mm

    dmin = jnp.min(dist, axis=1, keepdims=True)
    idx = jnp.min(jnp.where(dist == dmin, iota_m, m), axis=1,
                  keepdims=True)

    z = -jnp.sqrt(dist)
    zmax = -jnp.sqrt(dmin)
    ez = jnp.exp(z - zmax)
    p = ez / jnp.sum(ez, axis=1, keepdims=True)
    ent = -jnp.sum(p * jnp.log(p + 1e-05), axis=1, keepdims=True)
    adj = 1.0 - ent / max_ent

    ph_row = jnp.mean(p, axis=0, keepdims=True)
    ph[pl.ds(s * b_total + b, 1), :] = ph_row
    cons[s, 0] += jnp.sum(jnp.abs(p - ph_row))

    bf16 = jnp.bfloat16
    onehot = jnp.where(iota_m == idx, 1.0, 0.0).astype(bf16)
    q_ref[0, 0] = _dot(onehot, emb, (((1,), (0,))))
    ones_row = jnp.ones((1, t), bf16)
    counts[pl.ds(s, 1), :] += _dot(adj.astype(bf16), onehot, (((0,), (0,))))
    dwt[pl.ds(s * d, d), :] += _dot((adj * x_sum).astype(bf16), onehot,
                                    (((0,), (0,))))

    cnt = _dot(ones_row, onehot, (((1,), (0,))))
    cmax = jnp.max(cnt)
    iota_row = jax.lax.broadcasted_iota(jnp.int32, (1, m), 1)
    modes[s, b] = jnp.min(jnp.where(cnt == cmax, iota_row, m))

    @pl.when(jnp.logical_and(s == 1, b == b_total - 1))
    def _epilogue():
        ec = ec_ref[...]
        ewt = ewt_ref[...]
        counts_a = counts[pl.ds(0, 1), :]
        counts_v = counts[pl.ds(1, 1), :]
        c1 = DECAY * ec + (1.0 - DECAY) * counts_v
        n1 = jnp.sum(c1)
        c1 = (c1 + EPS) / (n1 + m * EPS) * n1
        w1 = DECAY * ewt + 0.5 * (1.0 - DECAY) * dwt[pl.ds(d, d), :]
        c2 = DECAY * c1 + (1.0 - DECAY) * counts_a
        n2 = jnp.sum(c2)
        c2 = (c2 + EPS) / (n2 + m * EPS) * n2
        w2 = DECAY * w1 + 0.5 * (1.0 - DECAY) * dwt[pl.ds(0, d), :]
        ent_ref[...] = w2 / c2

        pha = ph[pl.ds(0, b_total), :]
        phv = ph[pl.ds(b_total, b_total), :]
        la = jnp.log(pha + 1e-10)
        lv = jnp.log(phv + 1e-10)
        sc = (_dot(pha, lv, (((1,), (1,))), precision=_HI)
              + _dot(phv, la, (((1,), (1,))), precision=_HI))
        scmax = jnp.max(-sc)
        es = jnp.exp(sc + scmax)
        row = jax.lax.broadcasted_iota(jnp.int32, (b_total, b_total), 0)
        col = jax.lax.broadcasted_iota(jnp.int32, (b_total, b_total), 1)
        diag = jnp.sum(jnp.where(row == col, es, 0.0), axis=1, keepdims=True)
        es_sum = jnp.sum(es, axis=1, keepdims=True)
        lcmcm = -jnp.mean(jnp.log(diag / (es_sum + EPS)))

        eq = 0
        for i in range(b_total):
            eq += jnp.where(modes[0, i] == modes[1, i], 1, 0)

        lcm_ref[...] = jnp.full((1, 1), lcmcm, jnp.float32)
        ac_ref[...] = jnp.full((1, 1), cons[0, 0] / b_total, jnp.float32)
        vc_ref[...] = jnp.full((1, 1), cons[1, 0] / b_total, jnp.float32)
        eq_ref[...] = jnp.full((1, 1), eq, jnp.int32)


def kernel(audio_semantic, video_semantic, embedding, ema_count, ema_weight,
           epoch):
    b, t, d = audio_semantic.shape
    m = embedding.shape[0]

    fused = functools.partial(_fused_kernel, b_total=b, t=t, d=d, m=m)
    blk = lambda s, i: (i, 0, 0)
    full2 = lambda s, i: (0, 0)

    out_shape = [
        jax.ShapeDtypeStruct((2, b, t, d), jnp.float32),
        jax.ShapeDtypeStruct((d, m), jnp.float32),
        jax.ShapeDtypeStruct((1, 1), jnp.float32),
        jax.ShapeDtypeStruct((1, 1), jnp.float32),
        jax.ShapeDtypeStruct((1, 1), jnp.float32),
        jax.ShapeDtypeStruct((1, 1), jnp.int32),
    ]
    in_specs = [
        pl.BlockSpec((1, t, d), blk),
        pl.BlockSpec((1, t, d), blk),
        pl.BlockSpec((m, d), full2),
        pl.BlockSpec((d, m), full2),
        pl.BlockSpec((1, m), full2),
        pl.BlockSpec((d, m), full2),
    ]
    out_specs = [
        pl.BlockSpec((1, 1, t, d), lambda s, i: (s, i, 0, 0)),
        pl.BlockSpec((d, m), full2),
        pl.BlockSpec((1, 1), full2),
        pl.BlockSpec((1, 1), full2),
        pl.BlockSpec((1, 1), full2),
        pl.BlockSpec((1, 1), full2),
    ]
    scratch_shapes = [
        pltpu.VMEM((2, m), jnp.float32),
        pltpu.VMEM((2 * d, m), jnp.float32),
        pltpu.VMEM((2 * b, m), jnp.float32),
        pltpu.SMEM((2, 1), jnp.float32),
        pltpu.SMEM((2, b), jnp.int32),
        pltpu.VMEM((1, m), jnp.float32),
    ]

    outs = pl.pallas_call(
        fused,
        grid=(2, b),
        in_specs=in_specs,
        out_specs=out_specs,
        out_shape=out_shape,
        scratch_shapes=scratch_shapes,
        compiler_params=pltpu.CompilerParams(
            dimension_semantics=("arbitrary", "arbitrary"),
        ),
    )(
        audio_semantic, video_semantic, embedding.astype(jnp.bfloat16),
        embedding.T, ema_count.reshape(1, m), ema_weight.T,
    )
    q, ent_t, lcm, a_cons, v_cons, eq = outs
    return (q[0], q[1], ent_t.T, lcm[0, 0], a_cons[0, 0], v_cons[0, 0],
            eq[0, 0])


# speedup vs baseline: 2.7432x; 1.0274x over previous
import functools
import math

import jax
import jax.numpy as jnp
from jax.experimental import pallas as pl
from jax.experimental.pallas import tpu as pltpu

EPS = 1e-05
DECAY = 0.99

_HI = jax.lax.Precision.HIGHEST


def _dot(a, b, dims, precision=None):
    return jax.lax.dot_general(a, b, (dims, ((), ())), precision=precision,
                               preferred_element_type=jnp.float32)


def _fused_kernel(
    a_ref, v_ref, emb_ref, embt_ref, ec_ref, ewt_ref,
    q_ref, ent_ref, lcm_ref, ac_ref, vc_ref, eq_ref,
    counts, dwt, ph, cons, modes, esq_s,
    *, b_total, t, d, m,
):
    s = pl.program_id(0)
    b = pl.program_id(1)
    max_ent = math.log(m)

    @pl.when(jnp.logical_and(s == 0, b == 0))
    def _init():
        counts[...] = jnp.zeros_like(counts)
        dwt[...] = jnp.zeros_like(dwt)
        cons[0, 0] = 0.0
        cons[1, 0] = 0.0
        embt0 = embt_ref[...]
        esq_s[...] = jnp.sum(embt0 * embt0, axis=0, keepdims=True)

    emb = emb_ref[...]
    embt = embt_ref[...]
    esq = esq_s[...]
    iota_m = jax.lax.broadcasted_iota(jnp.int32, (t, m), 1)

    x_a = a_ref[0]
    x_v = v_ref[0]
    x_sum = x_a + x_v
    x = jnp.where(s == 0, x_a, x_v)

    xsq = jnp.sum(x * x, axis=1, keepdims=True)
    mm = _dot(x, embt, (((1,), (0,))))
    dist = (esq + xsq) - 2.0 * mm

    dmin = jnp.min(dist, axis=1, keepdims=True)
    idx = jnp.min(jnp.where(dist == dmin, iota_m, m), axis=1,
                  keepdims=True)

    z = -jnp.sqrt(dist)
    zmax = -jnp.sqrt(dmin)
    ez = jnp.exp(z - zmax)
    p = ez / jnp.sum(ez, axis=1, keepdims=True)
    ent = -jnp.sum(p * jnp.log(p + 1e-05), axis=1, keepdims=True)
    adj = 1.0 - ent / max_ent

    ph_row = jnp.mean(p, axis=0, keepdims=True)
    ph[pl.ds(s * b_total + b, 1), :] = ph_row
    cons[s, 0] += jnp.sum(jnp.abs(p - ph_row))

    bf16 = jnp.bfloat16
    onehot = jnp.where(iota_m == idx, 1.0, 0.0).astype(bf16)
    q_ref[0, 0] = _dot(onehot, emb, (((1,), (0,))))
    lhs = jnp.concatenate(
        [(adj * x_sum).astype(bf16), adj.astype(bf16),
         jnp.ones((t, 1), bf16)], axis=1)
    stats = _dot(lhs, onehot, (((0,), (0,))))
    dwt[pl.ds(s * d, d), :] += stats[0:d, :]
    counts[pl.ds(s, 1), :] += stats[d:d + 1, :]

    cnt = stats[d + 1:d + 2, :]
    cmax = jnp.max(cnt)
    iota_row = jax.lax.broadcasted_iota(jnp.int32, (1, m), 1)
    modes[s, b] = jnp.min(jnp.where(cnt == cmax, iota_row, m))

    @pl.when(jnp.logical_and(s == 1, b == b_total - 1))
    def _epilogue():
        ec = ec_ref[...]
        ewt = ewt_ref[...]
        counts_a = counts[pl.ds(0, 1), :]
        counts_v = counts[pl.ds(1, 1), :]
        c1 = DECAY * ec + (1.0 - DECAY) * counts_v
        n1 = jnp.sum(c1)
        c1 = (c1 + EPS) / (n1 + m * EPS) * n1
        w1 = DECAY * ewt + 0.5 * (1.0 - DECAY) * dwt[pl.ds(d, d), :]
        c2 = DECAY * c1 + (1.0 - DECAY) * counts_a
        n2 = jnp.sum(c2)
        c2 = (c2 + EPS) / (n2 + m * EPS) * n2
        w2 = DECAY * w1 + 0.5 * (1.0 - DECAY) * dwt[pl.ds(0, d), :]
        ent_ref[...] = w2 / c2

        pha = ph[pl.ds(0, b_total), :]
        phv = ph[pl.ds(b_total, b_total), :]
        la = jnp.log(pha + 1e-10)
        lv = jnp.log(phv + 1e-10)
        sc = (_dot(pha, lv, (((1,), (1,))), precision=_HI)
              + _dot(phv, la, (((1,), (1,))), precision=_HI))
        scmax = jnp.max(-sc)
        es = jnp.exp(sc + scmax)
        row = jax.lax.broadcasted_iota(jnp.int32, (b_total, b_total), 0)
        col = jax.lax.broadcasted_iota(jnp.int32, (b_total, b_total), 1)
        diag = jnp.sum(jnp.where(row == col, es, 0.0), axis=1, keepdims=True)
        es_sum = jnp.sum(es, axis=1, keepdims=True)
        lcmcm = -jnp.mean(jnp.log(diag / (es_sum + EPS)))

        eq = 0
        for i in range(b_total):
            eq += jnp.where(modes[0, i] == modes[1, i], 1, 0)

        lcm_ref[...] = jnp.full((1, 1), lcmcm, jnp.float32)
        ac_ref[...] = jnp.full((1, 1), cons[0, 0] / b_total, jnp.float32)
        vc_ref[...] = jnp.full((1, 1), cons[1, 0] / b_total, jnp.float32)
        eq_ref[...] = jnp.full((1, 1), eq, jnp.int32)


def kernel(audio_semantic, video_semantic, embedding, ema_count, ema_weight,
           epoch):
    b, t, d = audio_semantic.shape
    m = embedding.shape[0]

    fused = functools.partial(_fused_kernel, b_total=b, t=t, d=d, m=m)
    blk = lambda s, i: (i, 0, 0)
    full2 = lambda s, i: (0, 0)

    out_shape = [
        jax.ShapeDtypeStruct((2, b, t, d), jnp.float32),
        jax.ShapeDtypeStruct((d, m), jnp.float32),
        jax.ShapeDtypeStruct((1, 1), jnp.float32),
        jax.ShapeDtypeStruct((1, 1), jnp.float32),
        jax.ShapeDtypeStruct((1, 1), jnp.float32),
        jax.ShapeDtypeStruct((1, 1), jnp.int32),
    ]
    in_specs = [
        pl.BlockSpec((1, t, d), blk),
        pl.BlockSpec((1, t, d), blk),
        pl.BlockSpec((m, d), full2),
        pl.BlockSpec((d, m), full2),
        pl.BlockSpec((1, m), full2),
        pl.BlockSpec((d, m), full2),
    ]
    out_specs = [
        pl.BlockSpec((1, 1, t, d), lambda s, i: (s, i, 0, 0)),
        pl.BlockSpec((d, m), full2),
        pl.BlockSpec((1, 1), full2),
        pl.BlockSpec((1, 1), full2),
        pl.BlockSpec((1, 1), full2),
        pl.BlockSpec((1, 1), full2),
    ]
    scratch_shapes = [
        pltpu.VMEM((2, m), jnp.float32),
        pltpu.VMEM((2 * d, m), jnp.float32),
        pltpu.VMEM((2 * b, m), jnp.float32),
        pltpu.SMEM((2, 1), jnp.float32),
        pltpu.SMEM((2, b), jnp.int32),
        pltpu.VMEM((1, m), jnp.float32),
    ]

    outs = pl.pallas_call(
        fused,
        grid=(2, b),
        in_specs=in_specs,
        out_specs=out_specs,
        out_shape=out_shape,
        scratch_shapes=scratch_shapes,
        compiler_params=pltpu.CompilerParams(
            dimension_semantics=("arbitrary", "arbitrary"),
        ),
    )(
        audio_semantic, video_semantic, embedding.astype(jnp.bfloat16),
        embedding.T, ema_count.reshape(1, m), ema_weight.T,
    )
    q, ent_t, lcm, a_cons, v_cons, eq = outs
    return (q[0], q[1], ent_t.T, lcm[0, 0], a_cons[0, 0], v_cons[0, 0],
            eq[0, 0])


# speedup vs baseline: 3.0391x; 1.1079x over previous
import functools
import math

import jax
import jax.numpy as jnp
from jax.experimental import pallas as pl
from jax.experimental.pallas import tpu as pltpu

EPS = 1e-05
DECAY = 0.99

_HI = jax.lax.Precision.HIGHEST


def _dot(a, b, dims, precision=None):
    return jax.lax.dot_general(a, b, (dims, ((), ())), precision=precision,
                               preferred_element_type=jnp.float32)


def _fused_kernel(
    a_ref, v_ref, emb_ref, embt_ref, ec_ref, ewt_ref,
    q_ref, ent_ref, lcm_ref, ac_ref, vc_ref, eq_ref,
    counts, dwt, ph, cons, modes, esq_s,
    *, b_total, t, d, m,
):
    s = pl.program_id(0)
    b = pl.program_id(1)
    max_ent = math.log(m)

    @pl.when(jnp.logical_and(s == 0, b == 0))
    def _init():
        counts[...] = jnp.zeros_like(counts)
        dwt[...] = jnp.zeros_like(dwt)
        cons[0, 0] = 0.0
        cons[1, 0] = 0.0
        embt0 = embt_ref[...]
        esq_s[...] = jnp.sum(embt0 * embt0, axis=0, keepdims=True)

    emb = emb_ref[...]
    embt = embt_ref[...]
    esq = esq_s[...]
    iota_m = jax.lax.broadcasted_iota(jnp.int32, (t, m), 1)

    x_a = a_ref[0]
    x_v = v_ref[0]
    x_sum = x_a + x_v
    x = jnp.where(s == 0, x_a, x_v)

    xsq = jnp.sum(x * x, axis=1, keepdims=True)
    mm = _dot(x, embt, (((1,), (0,))))
    dist = (esq + xsq) - 2.0 * mm

    dmin = jnp.min(dist, axis=1, keepdims=True)
    idx = jnp.min(jnp.where(dist == dmin, iota_m, m), axis=1,
                  keepdims=True)

    z = -jnp.sqrt(dist)
    zmax = -jnp.sqrt(dmin)
    ez = jnp.exp(z - zmax)
    ssum = _dot(ez, jnp.ones((m, 1), jnp.float32), (((1,), (0,))))
    p = ez / ssum
    ent = -jnp.sum(p * jnp.log(p + 1e-05), axis=1, keepdims=True)
    adj = 1.0 - ent / max_ent

    ones_t = jnp.ones((1, t), jnp.float32)
    ph_row = _dot(ones_t, p, (((1,), (0,)))) * (1.0 / t)
    ph[pl.ds(s * b_total + b, 1), :] = ph_row
    absd = jnp.abs(p - ph_row)
    cons[s, 0] += jnp.sum(_dot(ones_t, absd, (((1,), (0,)))))

    bf16 = jnp.bfloat16
    onehot = jnp.where(iota_m == idx, 1.0, 0.0).astype(bf16)
    q_ref[0, 0] = _dot(onehot, emb, (((1,), (0,))))
    lhs = jnp.concatenate(
        [(adj * x_sum).astype(bf16), adj.astype(bf16),
         jnp.ones((t, 1), bf16)], axis=1)
    stats = _dot(lhs, onehot, (((0,), (0,))))
    dwt[pl.ds(s * d, d), :] += stats[0:d, :]
    counts[pl.ds(s, 1), :] += stats[d:d + 1, :]

    cnt = stats[d + 1:d + 2, :]
    cmax = jnp.max(cnt)
    iota_row = jax.lax.broadcasted_iota(jnp.int32, (1, m), 1)
    modes[s, b] = jnp.min(jnp.where(cnt == cmax, iota_row, m))

    @pl.when(jnp.logical_and(s == 1, b == b_total - 1))
    def _epilogue():
        ec = ec_ref[...]
        ewt = ewt_ref[...]
        counts_a = counts[pl.ds(0, 1), :]
        counts_v = counts[pl.ds(1, 1), :]
        c1 = DECAY * ec + (1.0 - DECAY) * counts_v
        n1 = jnp.sum(c1)
        c1 = (c1 + EPS) / (n1 + m * EPS) * n1
        w1 = DECAY * ewt + 0.5 * (1.0 - DECAY) * dwt[pl.ds(d, d), :]
        c2 = DECAY * c1 + (1.0 - DECAY) * counts_a
        n2 = jnp.sum(c2)
        c2 = (c2 + EPS) / (n2 + m * EPS) * n2
        w2 = DECAY * w1 + 0.5 * (1.0 - DECAY) * dwt[pl.ds(0, d), :]
        ent_ref[...] = w2 / c2

        pha = ph[pl.ds(0, b_total), :]
        phv = ph[pl.ds(b_total, b_total), :]
        la = jnp.log(pha + 1e-10)
        lv = jnp.log(phv + 1e-10)
        sc = (_dot(pha, lv, (((1,), (1,))), precision=_HI)
              + _dot(phv, la, (((1,), (1,))), precision=_HI))
        scmax = jnp.max(-sc)
        es = jnp.exp(sc + scmax)
        row = jax.lax.broadcasted_iota(jnp.int32, (b_total, b_total), 0)
        col = jax.lax.broadcasted_iota(jnp.int32, (b_total, b_total), 1)
        diag = jnp.sum(jnp.where(row == col, es, 0.0), axis=1, keepdims=True)
        es_sum = jnp.sum(es, axis=1, keepdims=True)
        lcmcm = -jnp.mean(jnp.log(diag / (es_sum + EPS)))

        eq = 0
        for i in range(b_total):
            eq += jnp.where(modes[0, i] == modes[1, i], 1, 0)

        lcm_ref[...] = jnp.full((1, 1), lcmcm, jnp.float32)
        ac_ref[...] = jnp.full((1, 1), cons[0, 0] / b_total, jnp.float32)
        vc_ref[...] = jnp.full((1, 1), cons[1, 0] / b_total, jnp.float32)
        eq_ref[...] = jnp.full((1, 1), eq, jnp.int32)


def kernel(audio_semantic, video_semantic, embedding, ema_count, ema_weight,
           epoch):
    b, t, d = audio_semantic.shape
    m = embedding.shape[0]

    fused = functools.partial(_fused_kernel, b_total=b, t=t, d=d, m=m)
    blk = lambda s, i: (i, 0, 0)
    full2 = lambda s, i: (0, 0)

    out_shape = [
        jax.ShapeDtypeStruct((2, b, t, d), jnp.float32),
        jax.ShapeDtypeStruct((d, m), jnp.float32),
        jax.ShapeDtypeStruct((1, 1), jnp.float32),
        jax.ShapeDtypeStruct((1, 1), jnp.float32),
        jax.ShapeDtypeStruct((1, 1), jnp.float32),
        jax.ShapeDtypeStruct((1, 1), jnp.int32),
    ]
    in_specs = [
        pl.BlockSpec((1, t, d), blk),
        pl.BlockSpec((1, t, d), blk),
        pl.BlockSpec((m, d), full2),
        pl.BlockSpec((d, m), full2),
        pl.BlockSpec((1, m), full2),
        pl.BlockSpec((d, m), full2),
    ]
    out_specs = [
        pl.BlockSpec((1, 1, t, d), lambda s, i: (s, i, 0, 0)),
        pl.BlockSpec((d, m), full2),
        pl.BlockSpec((1, 1), full2),
        pl.BlockSpec((1, 1), full2),
        pl.BlockSpec((1, 1), full2),
        pl.BlockSpec((1, 1), full2),
    ]
    scratch_shapes = [
        pltpu.VMEM((2, m), jnp.float32),
        pltpu.VMEM((2 * d, m), jnp.float32),
        pltpu.VMEM((2 * b, m), jnp.float32),
        pltpu.SMEM((2, 1), jnp.float32),
        pltpu.SMEM((2, b), jnp.int32),
        pltpu.VMEM((1, m), jnp.float32),
    ]

    outs = pl.pallas_call(
        fused,
        grid=(2, b),
        in_specs=in_specs,
        out_specs=out_specs,
        out_shape=out_shape,
        scratch_shapes=scratch_shapes,
        compiler_params=pltpu.CompilerParams(
            dimension_semantics=("arbitrary", "arbitrary"),
        ),
    )(
        audio_semantic, video_semantic, embedding.astype(jnp.bfloat16),
        embedding.T, ema_count.reshape(1, m), ema_weight.T,
    )
    q, ent_t, lcm, a_cons, v_cons, eq = outs
    return (q[0], q[1], ent_t.T, lcm[0, 0], a_cons[0, 0], v_cons[0, 0],
            eq[0, 0])
